# baseline (speedup 1.0000x reference)
import functools

import jax
import jax.numpy as jnp
from jax import lax
from jax.experimental import pallas as pl
from jax.experimental.pallas import tpu as pltpu
from jax.experimental.pallas import tpu_sc as plsc

N = 4096
NN = N * N
NNZ = 1_677_721

NC = 2
NS = 16
LANES = 16

C = 1_703_936
ROUNDS = 5
C_LAST = NN - (NC * ROUNDS - 1) * C

BLK = 2048
NBLK = 52
EPT = BLK * NBLK
NNZ_PAD = EPT * NS
PAD_IDX = 0x7F000000

NR = 32
ROW = 128
ZB = 4096
SLICE = C // NS
SLICE_LAST = C_LAST // NS
NZC = SLICE // ZB
NZC_LAST = SLICE_LAST // ZB
INFLIGHT_CAP = 12


def _sc_body(flat_hbm, vals_hbm, out_hbm, acc, idx_buf0, idx_buf1,
             val_buf0, val_buf1, stage_idx, stage_val, zeros_v,
             sem_in, sem_d, sem_z):
    c = lax.axis_index("c")
    s = lax.axis_index("s")
    tile_base = s * EPT

    zvec = jnp.zeros((LANES,), jnp.float32)

    def _fill_zeros(i, _):
        zeros_v[pl.ds(i * LANES, LANES)] = zvec
        return 0

    lax.fori_loop(0, ZB // LANES, _fill_zeros, 0)

    def _dummy_wait():
        pltpu.make_async_copy(
            vals_hbm.at[pl.ds(0, ROW)], stage_val.at[0], sem_d).wait()

    def _round(r, _):
        chunk = r * NC + c
        base = chunk * C
        is_last = chunk == NC * ROUNDS - 1
        bound = jnp.where(is_last, C_LAST, C)
        slice_sz = jnp.where(is_last, SLICE_LAST, SLICE)
        npieces = jnp.where(is_last, NZC_LAST, NZC)
        tslice = s * slice_sz

        def _zero(z, _):
            pltpu.async_copy(
                zeros_v, acc.at[pl.ds(tslice + z * ZB, ZB)], sem_z)
            return 0

        def _zero_wait(z, _):
            pltpu.make_async_copy(
                zeros_v, acc.at[pl.ds(tslice + z * ZB, ZB)], sem_z).wait()
            return 0

        lax.fori_loop(0, npieces, _zero, 0)
        lax.fori_loop(0, npieces, _zero_wait, 0)
        plsc.subcore_barrier()

        pltpu.async_copy(
            flat_hbm.at[pl.ds(tile_base, BLK)], idx_buf0, sem_in)
        pltpu.async_copy(
            vals_hbm.at[pl.ds(tile_base, BLK)], val_buf0, sem_in)

        def _scan_block(args, b, parity):
            cnt, drained, inflight = args
            ib = idx_buf0 if parity == 0 else idx_buf1
            vb = val_buf0 if parity == 0 else val_buf1
            nib = idx_buf1 if parity == 0 else idx_buf0
            nvb = val_buf1 if parity == 0 else val_buf0
            off_b = tile_base + b * BLK
            pltpu.make_async_copy(
                flat_hbm.at[pl.ds(off_b, BLK)], ib, sem_in).wait()
            pltpu.make_async_copy(
                vals_hbm.at[pl.ds(off_b, BLK)], vb, sem_in).wait()

            @pl.when(b + 1 < NBLK)
            def _():
                off = tile_base + (b + 1) * BLK
                pltpu.async_copy(flat_hbm.at[pl.ds(off, BLK)], nib, sem_in)
                pltpu.async_copy(vals_hbm.at[pl.ds(off, BLK)], nvb, sem_in)

            base_vec = jnp.zeros((LANES,), jnp.int32) + base
            cbound = plsc.bitcast(
                jnp.zeros((LANES,), jnp.int32) + bound, jnp.uint32)

            def _vec(v, cnt_vec):
                fl = ib[pl.ds(v * LANES, LANES)]
                vv = vb[pl.ds(v * LANES, LANES)]
                t = fl - base_vec
                mask = plsc.bitcast(t, jnp.uint32) < cbound
                cums = plsc.cumsum(mask.astype(jnp.int32))
                pos = cnt_vec + cums - 1
                rows = (pos >> 7) & (NR - 1)
                cols = pos & (ROW - 1)
                plsc.store_scatter(stage_idx, [rows, cols], t, mask=mask)
                plsc.store_scatter(stage_val, [rows, cols], vv, mask=mask)
                pc = plsc.all_reduce_population_count(mask)
                return cnt_vec + pc

            cnt_vec0 = jnp.zeros((LANES,), jnp.int32) + cnt
            cnt_vec = plsc.parallel_loop(
                0, BLK // LANES, 1, unroll=4, carry=cnt_vec0)(_vec)
            cnt = cnt_vec[0]

            full = cnt >> 7

            def _drain(j, _):
                row = j & (NR - 1)
                pltpu.async_copy(
                    stage_val.at[row], acc.at[stage_idx.at[row]], sem_d,
                    add=True)
                return 0

            lax.fori_loop(drained, full, _drain, 0)
            inflight = inflight + (full - drained)

            def _wait_one(j, _):
                _dummy_wait()
                return 0

            nwait = jnp.maximum(inflight - INFLIGHT_CAP, 0)
            lax.fori_loop(0, nwait, _wait_one, 0)
            inflight = inflight - nwait
            return cnt, full, inflight

        def _block_pair(i, args):
            args = _scan_block(args, 2 * i, 0)
            args = _scan_block(args, 2 * i + 1, 1)
            return args

        cnt, drained, inflight = lax.fori_loop(
            0, NBLK // 2, _block_pair, (jnp.int32(0), jnp.int32(0),
                                        jnp.int32(0)))

        col = cnt & (ROW - 1)
        row_last = (cnt >> 7) & (NR - 1)

        @pl.when(col != 0)
        def _():
            rvec = jnp.zeros((LANES,), jnp.int32) + row_last
            zidx = jnp.zeros((LANES,), jnp.int32)
            cvec = jnp.zeros((LANES,), jnp.int32) + col
            for k in range(ROW // LANES):
                lane = lax.iota(jnp.int32, LANES) + (k * LANES) + cvec
                m = lane < ROW
                plsc.store_scatter(stage_idx, [rvec, lane], zidx, mask=m)
                plsc.store_scatter(stage_val, [rvec, lane], zvec, mask=m)

        full_end = (cnt + (ROW - 1)) >> 7

        def _drain_tail(j, _):
            row = j & (NR - 1)
            pltpu.async_copy(
                stage_val.at[row], acc.at[stage_idx.at[row]], sem_d, add=True)
            return 0

        lax.fori_loop(drained, full_end, _drain_tail, 0)
        inflight = inflight + (full_end - drained)

        def _wait_all(j, _):
            _dummy_wait()
            return 0

        lax.fori_loop(0, inflight, _wait_all, 0)
        plsc.subcore_barrier()

        def _flush(z, _):
            pltpu.async_copy(
                acc.at[pl.ds(tslice + z * ZB, ZB)],
                out_hbm.at[pl.ds(base + tslice + z * ZB, ZB)], sem_z)
            return 0

        def _flush_wait(z, _):
            pltpu.make_async_copy(
                acc.at[pl.ds(tslice + z * ZB, ZB)],
                out_hbm.at[pl.ds(base + tslice + z * ZB, ZB)], sem_z).wait()
            return 0

        lax.fori_loop(0, npieces, _flush, 0)
        lax.fori_loop(0, npieces, _flush_wait, 0)
        plsc.subcore_barrier()
        return 0

    lax.fori_loop(0, ROUNDS, _round, 0)


_densify_call = pl.kernel(
    _sc_body,
    out_type=jax.ShapeDtypeStruct((NN,), jnp.float32),
    mesh=plsc.VectorSubcoreMesh(
        core_axis_name="c", subcore_axis_name="s", num_cores=NC,
        num_subcores=NS),
    scratch_types=[
        pltpu.VMEM_SHARED((C,), jnp.float32),
        pltpu.VMEM((BLK,), jnp.int32),
        pltpu.VMEM((BLK,), jnp.int32),
        pltpu.VMEM((BLK,), jnp.float32),
        pltpu.VMEM((BLK,), jnp.float32),
        pltpu.VMEM((NR, ROW), jnp.int32),
        pltpu.VMEM((NR, ROW), jnp.float32),
        pltpu.VMEM((ZB,), jnp.float32),
        pltpu.SemaphoreType.DMA,
        pltpu.SemaphoreType.DMA,
        pltpu.SemaphoreType.DMA,
    ],
    compiler_params=pltpu.CompilerParams(needs_layout_passes=False),
    cost_estimate=pl.CostEstimate(
        flops=2 * NNZ_PAD * ROUNDS,
        bytes_accessed=8 * NNZ_PAD * ROUNDS * NC + 4 * NN,
        transcendentals=0),
)


def _build_wt(rows, cols, vals):
    flat = cols * N + rows
    flat = jnp.concatenate(
        [flat, jnp.full((NNZ_PAD - NNZ,), PAD_IDX, jnp.int32)])
    v = jnp.concatenate(
        [vals, jnp.zeros((NNZ_PAD - NNZ,), jnp.float32)])
    return _densify_call(flat, v).reshape(N, N)


BM = 1024
BN = 512
BK = 512


def _mm_body(apply_tanh, out_dtype, x_ref, w_ref, b_ref, o_ref, acc_ref):
    k = pl.program_id(1)

    @pl.when(k == 0)
    def _():
        acc_ref[...] = jnp.zeros_like(acc_ref)

    acc_ref[...] += jnp.dot(
        x_ref[...], w_ref[...].astype(jnp.bfloat16),
        preferred_element_type=jnp.float32)

    @pl.when(k == pl.num_programs(1) - 1)
    def _():
        y = acc_ref[...] + b_ref[...].astype(jnp.float32)
        if apply_tanh:
            y = jnp.tanh(y)
        o_ref[...] = y.astype(out_dtype)


def _mm(x_bf16, wt, bias, apply_tanh, out_dtype):
    b2 = bias.reshape(1, N)
    grid = (N // BN, N // BK)
    return pl.pallas_call(
        functools.partial(_mm_body, apply_tanh, out_dtype),
        grid=grid,
        in_specs=[
            pl.BlockSpec((BM, BK), lambda n, k: (0, k)),
            pl.BlockSpec((BK, BN), lambda n, k: (k, n)),
            pl.BlockSpec((1, BN), lambda n, k: (0, n)),
        ],
        out_specs=pl.BlockSpec((BM, BN), lambda n, k: (0, n)),
        out_shape=jax.ShapeDtypeStruct((BM, N), out_dtype),
        scratch_shapes=[pltpu.VMEM((BM, BN), jnp.float32)],
        compiler_params=pltpu.CompilerParams(
            dimension_semantics=("parallel", "arbitrary")),
    )(x_bf16, wt, b2)


def kernel(x, rows0, cols0, vals0, bias0, rows1, cols1, vals1, bias1,
           rows2, cols2, vals2, bias2):
    wt0 = _build_wt(rows0, cols0, vals0)
    wt1 = _build_wt(rows1, cols1, vals1)
    wt2 = _build_wt(rows2, cols2, vals2)
    h = _mm(x.astype(jnp.bfloat16), wt0, bias0, True, jnp.bfloat16)
    h = _mm(h, wt1, bias1, True, jnp.bfloat16)
    return _mm(h, wt2, bias2, False, jnp.float32)


# speedup vs baseline: 1.0168x; 1.0168x over previous
import functools

import jax
import jax.numpy as jnp
from jax import lax
from jax.experimental import pallas as pl
from jax.experimental.pallas import tpu as pltpu
from jax.experimental.pallas import tpu_sc as plsc

N = 4096
NN = N * N
NNZ = 1_677_721

NC = 2
NS = 16
LANES = 16

C = 1_703_936
ROUNDS = 5
C_LAST = NN - (NC * ROUNDS - 1) * C

BLK = 2048
NBLK = 52
EPT = BLK * NBLK
NNZ_PAD = EPT * NS
PAD_IDX = 0x7F000000

NR = 32
ROW = 128
ZB = 4096
SLICE = C // NS
SLICE_LAST = C_LAST // NS
NZC = SLICE // ZB
NZC_LAST = SLICE_LAST // ZB
INFLIGHT_CAP = 12


def _sc_body(flat_hbm, vals_hbm, out_hbm, acc, idx_buf0, idx_buf1,
             val_buf0, val_buf1, stage_idx, stage_val, zeros_v,
             sem_in, sem_d, sem_z):
    c = lax.axis_index("c")
    s = lax.axis_index("s")
    tile_base = s * EPT

    zvec = jnp.zeros((LANES,), jnp.float32)

    def _fill_zeros(i, _):
        zeros_v[pl.ds(i * LANES, LANES)] = zvec
        return 0

    lax.fori_loop(0, ZB // LANES, _fill_zeros, 0)

    def _dummy_wait():
        pltpu.make_async_copy(
            vals_hbm.at[pl.ds(0, ROW)], stage_val.at[0], sem_d).wait()

    def _round(r, _):
        chunk = r * NC + c
        base = chunk * C
        is_last = chunk == NC * ROUNDS - 1
        bound = jnp.where(is_last, C_LAST, C)
        slice_sz = jnp.where(is_last, SLICE_LAST, SLICE)
        npieces = jnp.where(is_last, NZC_LAST, NZC)
        tslice = s * slice_sz

        def _zero(z, _):
            pltpu.async_copy(
                zeros_v, acc.at[pl.ds(tslice + z * ZB, ZB)], sem_z)
            return 0

        def _zero_wait(z, _):
            pltpu.make_async_copy(
                zeros_v, acc.at[pl.ds(tslice + z * ZB, ZB)], sem_z).wait()
            return 0

        lax.fori_loop(0, npieces, _zero, 0)
        lax.fori_loop(0, npieces, _zero_wait, 0)
        plsc.subcore_barrier()

        pltpu.async_copy(
            flat_hbm.at[pl.ds(tile_base, BLK)], idx_buf0, sem_in)
        pltpu.async_copy(
            vals_hbm.at[pl.ds(tile_base, BLK)], val_buf0, sem_in)

        def _scan_block(args, b, parity):
            cnt, drained, inflight = args
            ib = idx_buf0 if parity == 0 else idx_buf1
            vb = val_buf0 if parity == 0 else val_buf1
            nib = idx_buf1 if parity == 0 else idx_buf0
            nvb = val_buf1 if parity == 0 else val_buf0
            off_b = tile_base + b * BLK
            pltpu.make_async_copy(
                flat_hbm.at[pl.ds(off_b, BLK)], ib, sem_in).wait()
            pltpu.make_async_copy(
                vals_hbm.at[pl.ds(off_b, BLK)], vb, sem_in).wait()

            @pl.when(b + 1 < NBLK)
            def _():
                off = tile_base + (b + 1) * BLK
                pltpu.async_copy(flat_hbm.at[pl.ds(off, BLK)], nib, sem_in)
                pltpu.async_copy(vals_hbm.at[pl.ds(off, BLK)], nvb, sem_in)

            base_vec = jnp.zeros((LANES,), jnp.int32) + base
            cbound = plsc.bitcast(
                jnp.zeros((LANES,), jnp.int32) + bound, jnp.uint32)

            def _vec(v, cnt_vec):
                fl = ib[pl.ds(v * LANES, LANES)]
                vv = vb[pl.ds(v * LANES, LANES)]
                t = fl - base_vec
                mask = plsc.bitcast(t, jnp.uint32) < cbound
                cums = plsc.cumsum(mask.astype(jnp.int32))
                pos = cnt_vec + cums
                rows = (pos >> 7) & (NR - 1)
                cols = pos & (ROW - 1)
                plsc.store_scatter(stage_idx, [rows, cols], t, mask=mask)
                plsc.store_scatter(stage_val, [rows, cols], vv, mask=mask)
                pc = plsc.all_reduce_population_count(mask)
                return cnt_vec + pc

            cnt_vec0 = jnp.zeros((LANES,), jnp.int32) + (cnt - 1)
            cnt_vec = plsc.parallel_loop(
                0, BLK // LANES, 1, unroll=4, carry=cnt_vec0)(_vec)
            cnt = cnt_vec[0] + 1

            full = cnt >> 7

            def _drain(j, _):
                row = j & (NR - 1)
                pltpu.async_copy(
                    stage_val.at[row], acc.at[stage_idx.at[row]], sem_d,
                    add=True)
                return 0

            lax.fori_loop(drained, full, _drain, 0)
            inflight = inflight + (full - drained)

            def _wait_one(j, _):
                _dummy_wait()
                return 0

            nwait = jnp.maximum(inflight - INFLIGHT_CAP, 0)
            lax.fori_loop(0, nwait, _wait_one, 0)
            inflight = inflight - nwait
            return cnt, full, inflight

        def _block_pair(i, args):
            args = _scan_block(args, 2 * i, 0)
            args = _scan_block(args, 2 * i + 1, 1)
            return args

        cnt, drained, inflight = lax.fori_loop(
            0, NBLK // 2, _block_pair, (jnp.int32(0), jnp.int32(0),
                                        jnp.int32(0)))

        col = cnt & (ROW - 1)
        row_last = (cnt >> 7) & (NR - 1)

        @pl.when(col != 0)
        def _():
            rvec = jnp.zeros((LANES,), jnp.int32) + row_last
            zidx = jnp.zeros((LANES,), jnp.int32)
            cvec = jnp.zeros((LANES,), jnp.int32) + col
            for k in range(ROW // LANES):
                lane = lax.iota(jnp.int32, LANES) + (k * LANES) + cvec
                m = lane < ROW
                plsc.store_scatter(stage_idx, [rvec, lane], zidx, mask=m)
                plsc.store_scatter(stage_val, [rvec, lane], zvec, mask=m)

        full_end = (cnt + (ROW - 1)) >> 7

        def _drain_tail(j, _):
            row = j & (NR - 1)
            pltpu.async_copy(
                stage_val.at[row], acc.at[stage_idx.at[row]], sem_d, add=True)
            return 0

        lax.fori_loop(drained, full_end, _drain_tail, 0)
        inflight = inflight + (full_end - drained)

        def _wait_all(j, _):
            _dummy_wait()
            return 0

        lax.fori_loop(0, inflight, _wait_all, 0)
        plsc.subcore_barrier()

        def _flush(z, _):
            pltpu.async_copy(
                acc.at[pl.ds(tslice + z * ZB, ZB)],
                out_hbm.at[pl.ds(base + tslice + z * ZB, ZB)], sem_z)
            return 0

        def _flush_wait(z, _):
            pltpu.make_async_copy(
                acc.at[pl.ds(tslice + z * ZB, ZB)],
                out_hbm.at[pl.ds(base + tslice + z * ZB, ZB)], sem_z).wait()
            return 0

        lax.fori_loop(0, npieces, _flush, 0)
        lax.fori_loop(0, npieces, _flush_wait, 0)
        plsc.subcore_barrier()
        return 0

    lax.fori_loop(0, ROUNDS, _round, 0)


_densify_call = pl.kernel(
    _sc_body,
    out_type=jax.ShapeDtypeStruct((NN,), jnp.float32),
    mesh=plsc.VectorSubcoreMesh(
        core_axis_name="c", subcore_axis_name="s", num_cores=NC,
        num_subcores=NS),
    scratch_types=[
        pltpu.VMEM_SHARED((C,), jnp.float32),
        pltpu.VMEM((BLK,), jnp.int32),
        pltpu.VMEM((BLK,), jnp.int32),
        pltpu.VMEM((BLK,), jnp.float32),
        pltpu.VMEM((BLK,), jnp.float32),
        pltpu.VMEM((NR, ROW), jnp.int32),
        pltpu.VMEM((NR, ROW), jnp.float32),
        pltpu.VMEM((ZB,), jnp.float32),
        pltpu.SemaphoreType.DMA,
        pltpu.SemaphoreType.DMA,
        pltpu.SemaphoreType.DMA,
    ],
    compiler_params=pltpu.CompilerParams(needs_layout_passes=False),
    cost_estimate=pl.CostEstimate(
        flops=2 * NNZ_PAD * ROUNDS,
        bytes_accessed=8 * NNZ_PAD * ROUNDS * NC + 4 * NN,
        transcendentals=0),
)


def _build_wt(rows, cols, vals):
    flat = cols * N + rows
    flat = jnp.concatenate(
        [flat, jnp.full((NNZ_PAD - NNZ,), PAD_IDX, jnp.int32)])
    v = jnp.concatenate(
        [vals, jnp.zeros((NNZ_PAD - NNZ,), jnp.float32)])
    return _densify_call(flat, v).reshape(N, N)


BM = 1024
BN = 512
BK = 1024


def _mm_body(apply_tanh, out_dtype, x_ref, w_ref, b_ref, o_ref, acc_ref):
    k = pl.program_id(1)

    @pl.when(k == 0)
    def _():
        acc_ref[...] = jnp.zeros_like(acc_ref)

    acc_ref[...] += jnp.dot(
        x_ref[...], w_ref[...].astype(jnp.bfloat16),
        preferred_element_type=jnp.float32)

    @pl.when(k == pl.num_programs(1) - 1)
    def _():
        y = acc_ref[...] + b_ref[...].astype(jnp.float32)
        if apply_tanh:
            y = jnp.tanh(y)
        o_ref[...] = y.astype(out_dtype)


def _mm(x_bf16, wt, bias, apply_tanh, out_dtype):
    b2 = bias.reshape(1, N)
    grid = (N // BN, N // BK)
    return pl.pallas_call(
        functools.partial(_mm_body, apply_tanh, out_dtype),
        grid=grid,
        in_specs=[
            pl.BlockSpec((BM, BK), lambda n, k: (0, k)),
            pl.BlockSpec((BK, BN), lambda n, k: (k, n)),
            pl.BlockSpec((1, BN), lambda n, k: (0, n)),
        ],
        out_specs=pl.BlockSpec((BM, BN), lambda n, k: (0, n)),
        out_shape=jax.ShapeDtypeStruct((BM, N), out_dtype),
        scratch_shapes=[pltpu.VMEM((BM, BN), jnp.float32)],
        compiler_params=pltpu.CompilerParams(
            dimension_semantics=("parallel", "arbitrary")),
    )(x_bf16, wt, b2)


def kernel(x, rows0, cols0, vals0, bias0, rows1, cols1, vals1, bias1,
           rows2, cols2, vals2, bias2):
    wt0 = _build_wt(rows0, cols0, vals0)
    wt1 = _build_wt(rows1, cols1, vals1)
    wt2 = _build_wt(rows2, cols2, vals2)
    h = _mm(x.astype(jnp.bfloat16), wt0, bias0, True, jnp.bfloat16)
    h = _mm(h, wt1, bias1, True, jnp.bfloat16)
    return _mm(h, wt2, bias2, False, jnp.float32)


# speedup vs baseline: 1.0372x; 1.0201x over previous
import functools

import jax
import jax.numpy as jnp
from jax import lax
from jax.experimental import pallas as pl
from jax.experimental.pallas import tpu as pltpu
from jax.experimental.pallas import tpu_sc as plsc

N = 4096
NN = N * N
NNZ = 1_677_721

NC = 2
NS = 16
LANES = 16

C = 1_703_936
ROUNDS = 5
C_LAST = NN - (NC * ROUNDS - 1) * C

BLK = 2048
NBLK = 52
EPT = BLK * NBLK
NNZ_PAD = EPT * NS
PAD_IDX = 0x7F000000

NR = 32
ROW = 128
ZB = 4096
SLICE = C // NS
SLICE_LAST = C_LAST // NS
NZC = SLICE // ZB
NZC_LAST = SLICE_LAST // ZB
INFLIGHT_CAP = 12


def _sc_body(flat_hbm, vals_hbm, out_hbm, acc, idx_buf0, idx_buf1,
             val_buf0, val_buf1, stage_idx, stage_val, zeros_v,
             sem_in, sem_d, sem_z):
    c = lax.axis_index("c")
    s = lax.axis_index("s")
    tile_base = s * EPT

    zvec = jnp.zeros((LANES,), jnp.float32)

    def _fill_zeros(i, _):
        zeros_v[pl.ds(i * LANES, LANES)] = zvec
        return 0

    lax.fori_loop(0, ZB // LANES, _fill_zeros, 0)

    def _dummy_wait():
        pltpu.make_async_copy(
            vals_hbm.at[pl.ds(0, ROW)], stage_val.at[0], sem_d).wait()

    def _round(r, _):
        chunk = r * NC + c
        base = chunk * C
        is_last = chunk == NC * ROUNDS - 1
        bound = jnp.where(is_last, C_LAST, C)
        slice_sz = jnp.where(is_last, SLICE_LAST, SLICE)
        npieces = jnp.where(is_last, NZC_LAST, NZC)
        tslice = s * slice_sz

        def _zero(z, _):
            pltpu.async_copy(
                zeros_v, acc.at[pl.ds(tslice + z * ZB, ZB)], sem_z)
            return 0

        def _zero_wait(z, _):
            pltpu.make_async_copy(
                zeros_v, acc.at[pl.ds(tslice + z * ZB, ZB)], sem_z).wait()
            return 0

        lax.fori_loop(0, npieces, _zero, 0)
        lax.fori_loop(0, npieces, _zero_wait, 0)
        plsc.subcore_barrier()

        pltpu.async_copy(
            flat_hbm.at[pl.ds(tile_base, BLK)], idx_buf0, sem_in)
        pltpu.async_copy(
            vals_hbm.at[pl.ds(tile_base, BLK)], val_buf0, sem_in)

        def _scan_block(args, b, parity):
            cnt, drained, inflight = args
            ib = idx_buf0 if parity == 0 else idx_buf1
            vb = val_buf0 if parity == 0 else val_buf1
            nib = idx_buf1 if parity == 0 else idx_buf0
            nvb = val_buf1 if parity == 0 else val_buf0
            off_b = tile_base + b * BLK
            pltpu.make_async_copy(
                flat_hbm.at[pl.ds(off_b, BLK)], ib, sem_in).wait()
            pltpu.make_async_copy(
                vals_hbm.at[pl.ds(off_b, BLK)], vb, sem_in).wait()

            @pl.when(b + 1 < NBLK)
            def _():
                off = tile_base + (b + 1) * BLK
                pltpu.async_copy(flat_hbm.at[pl.ds(off, BLK)], nib, sem_in)
                pltpu.async_copy(vals_hbm.at[pl.ds(off, BLK)], nvb, sem_in)

            base_vec = jnp.zeros((LANES,), jnp.int32) + base
            cbound = plsc.bitcast(
                jnp.zeros((LANES,), jnp.int32) + bound, jnp.uint32)

            def _vec(v, cnt_vec):
                fl = ib[pl.ds(v * LANES, LANES)]
                vv = vb[pl.ds(v * LANES, LANES)]
                t = fl - base_vec
                mask = plsc.bitcast(t, jnp.uint32) < cbound
                cums = plsc.cumsum(mask.astype(jnp.int32))
                pos = cnt_vec + cums
                rows = (pos >> 7) & (NR - 1)
                cols = pos & (ROW - 1)
                plsc.store_scatter(stage_idx, [rows, cols], t, mask=mask)
                plsc.store_scatter(stage_val, [rows, cols], vv, mask=mask)
                pc = plsc.all_reduce_population_count(mask)
                return cnt_vec + pc

            cnt_vec0 = jnp.zeros((LANES,), jnp.int32) + (cnt - 1)
            cnt_vec = plsc.parallel_loop(
                0, BLK // LANES, 1, unroll=4, carry=cnt_vec0)(_vec)
            cnt = cnt_vec[0] + 1

            full = cnt >> 7

            def _drain(j, _):
                row = j & (NR - 1)
                pltpu.async_copy(
                    stage_val.at[row], acc.at[stage_idx.at[row]], sem_d,
                    add=True)
                return 0

            lax.fori_loop(drained, full, _drain, 0)
            inflight = inflight + (full - drained)

            def _wait_one(j, _):
                _dummy_wait()
                return 0

            nwait = jnp.maximum(inflight - INFLIGHT_CAP, 0)
            lax.fori_loop(0, nwait, _wait_one, 0)
            inflight = inflight - nwait
            return cnt, full, inflight

        def _block_pair(i, args):
            args = _scan_block(args, 2 * i, 0)
            args = _scan_block(args, 2 * i + 1, 1)
            return args

        cnt, drained, inflight = lax.fori_loop(
            0, NBLK // 2, _block_pair, (jnp.int32(0), jnp.int32(0),
                                        jnp.int32(0)))

        col = cnt & (ROW - 1)
        row_last = (cnt >> 7) & (NR - 1)

        @pl.when(col != 0)
        def _():
            rvec = jnp.zeros((LANES,), jnp.int32) + row_last
            zidx = jnp.zeros((LANES,), jnp.int32)
            cvec = jnp.zeros((LANES,), jnp.int32) + col
            for k in range(ROW // LANES):
                lane = lax.iota(jnp.int32, LANES) + (k * LANES) + cvec
                m = lane < ROW
                plsc.store_scatter(stage_idx, [rvec, lane], zidx, mask=m)
                plsc.store_scatter(stage_val, [rvec, lane], zvec, mask=m)

        full_end = (cnt + (ROW - 1)) >> 7

        def _drain_tail(j, _):
            row = j & (NR - 1)
            pltpu.async_copy(
                stage_val.at[row], acc.at[stage_idx.at[row]], sem_d, add=True)
            return 0

        lax.fori_loop(drained, full_end, _drain_tail, 0)
        inflight = inflight + (full_end - drained)

        def _wait_all(j, _):
            _dummy_wait()
            return 0

        lax.fori_loop(0, inflight, _wait_all, 0)
        plsc.subcore_barrier()

        def _flush(z, _):
            pltpu.async_copy(
                acc.at[pl.ds(tslice + z * ZB, ZB)],
                out_hbm.at[pl.ds(base + tslice + z * ZB, ZB)], sem_z)
            return 0

        def _flush_wait(z, _):
            pltpu.make_async_copy(
                acc.at[pl.ds(tslice + z * ZB, ZB)],
                out_hbm.at[pl.ds(base + tslice + z * ZB, ZB)], sem_z).wait()
            return 0

        lax.fori_loop(0, npieces, _flush, 0)
        lax.fori_loop(0, npieces, _flush_wait, 0)
        plsc.subcore_barrier()
        return 0

    lax.fori_loop(0, ROUNDS, _round, 0)


_densify_call = pl.kernel(
    _sc_body,
    out_type=jax.ShapeDtypeStruct((NN,), jnp.float32),
    mesh=plsc.VectorSubcoreMesh(
        core_axis_name="c", subcore_axis_name="s", num_cores=NC,
        num_subcores=NS),
    scratch_types=[
        pltpu.VMEM_SHARED((C,), jnp.float32),
        pltpu.VMEM((BLK,), jnp.int32),
        pltpu.VMEM((BLK,), jnp.int32),
        pltpu.VMEM((BLK,), jnp.float32),
        pltpu.VMEM((BLK,), jnp.float32),
        pltpu.VMEM((NR, ROW), jnp.int32),
        pltpu.VMEM((NR, ROW), jnp.float32),
        pltpu.VMEM((ZB,), jnp.float32),
        pltpu.SemaphoreType.DMA,
        pltpu.SemaphoreType.DMA,
        pltpu.SemaphoreType.DMA,
    ],
    compiler_params=pltpu.CompilerParams(needs_layout_passes=False),
    cost_estimate=pl.CostEstimate(
        flops=2 * NNZ_PAD * ROUNDS,
        bytes_accessed=8 * NNZ_PAD * ROUNDS * NC + 4 * NN,
        transcendentals=0),
)


def _build_wt(rows, cols, vals):
    flat = cols * N + rows
    flat = jnp.concatenate(
        [flat, jnp.full((NNZ_PAD - NNZ,), PAD_IDX, jnp.int32)])
    v = jnp.concatenate(
        [vals, jnp.zeros((NNZ_PAD - NNZ,), jnp.float32)])
    return _densify_call(flat, v).reshape(N, N)


BM = 1024
BN = 1024
BK = 1024


def _mm_body(apply_tanh, out_dtype, x_ref, w_ref, b_ref, o_ref, acc_ref):
    k = pl.program_id(1)

    @pl.when(k == 0)
    def _():
        acc_ref[...] = jnp.zeros_like(acc_ref)

    acc_ref[...] += jnp.dot(
        x_ref[...], w_ref[...].astype(jnp.bfloat16),
        preferred_element_type=jnp.float32)

    @pl.when(k == pl.num_programs(1) - 1)
    def _():
        y = acc_ref[...] + b_ref[...].astype(jnp.float32)
        if apply_tanh:
            y = jnp.tanh(y)
        o_ref[...] = y.astype(out_dtype)


def _mm(x_bf16, wt, bias, apply_tanh, out_dtype):
    b2 = bias.reshape(1, N)
    grid = (N // BN, N // BK)
    return pl.pallas_call(
        functools.partial(_mm_body, apply_tanh, out_dtype),
        grid=grid,
        in_specs=[
            pl.BlockSpec((BM, BK), lambda n, k: (0, k)),
            pl.BlockSpec((BK, BN), lambda n, k: (k, n)),
            pl.BlockSpec((1, BN), lambda n, k: (0, n)),
        ],
        out_specs=pl.BlockSpec((BM, BN), lambda n, k: (0, n)),
        out_shape=jax.ShapeDtypeStruct((BM, N), out_dtype),
        scratch_shapes=[pltpu.VMEM((BM, BN), jnp.float32)],
        compiler_params=pltpu.CompilerParams(
            dimension_semantics=("parallel", "arbitrary")),
    )(x_bf16, wt, b2)


def kernel(x, rows0, cols0, vals0, bias0, rows1, cols1, vals1, bias1,
           rows2, cols2, vals2, bias2):
    wt0 = _build_wt(rows0, cols0, vals0)
    wt1 = _build_wt(rows1, cols1, vals1)
    wt2 = _build_wt(rows2, cols2, vals2)
    h = _mm(x.astype(jnp.bfloat16), wt0, bias0, True, jnp.bfloat16)
    h = _mm(h, wt1, bias1, True, jnp.bfloat16)
    return _mm(h, wt2, bias2, False, jnp.float32)


# speedup vs baseline: 1.0400x; 1.0026x over previous
import functools

import jax
import jax.numpy as jnp
from jax import lax
from jax.experimental import pallas as pl
from jax.experimental.pallas import tpu as pltpu
from jax.experimental.pallas import tpu_sc as plsc

N = 4096
NN = N * N
NNZ = 1_677_721

NC = 2
NS = 16
LANES = 16

C = 1_703_936
ROUNDS = 5
C_LAST = NN - (NC * ROUNDS - 1) * C

BLK = 2048
NBLK = 52
EPT = BLK * NBLK
NNZ_PAD = EPT * NS
PAD_IDX = 0x7F000000

NR = 32
ROW = 128
ZB = 4096
SLICE = C // NS
SLICE_LAST = C_LAST // NS
NZC = SLICE // ZB
NZC_LAST = SLICE_LAST // ZB
INFLIGHT_CAP = 12


def _sc_body(flat_hbm, vals_hbm, out_hbm, acc, idx_buf0, idx_buf1,
             val_buf0, val_buf1, stage_idx, stage_val, zeros_v,
             sem_in, sem_d, sem_z):
    c = lax.axis_index("c")
    s = lax.axis_index("s")
    tile_base = s * EPT

    zvec = jnp.zeros((LANES,), jnp.float32)

    def _fill_zeros(i, _):
        zeros_v[pl.ds(i * LANES, LANES)] = zvec
        return 0

    lax.fori_loop(0, ZB // LANES, _fill_zeros, 0)

    def _dummy_wait():
        pltpu.make_async_copy(
            vals_hbm.at[pl.ds(0, ROW)], stage_val.at[0], sem_d).wait()

    def _round(r, _):
        chunk = r * NC + c
        base = chunk * C
        is_last = chunk == NC * ROUNDS - 1
        bound = jnp.where(is_last, C_LAST, C)
        slice_sz = jnp.where(is_last, SLICE_LAST, SLICE)
        npieces = jnp.where(is_last, NZC_LAST, NZC)
        tslice = s * slice_sz

        def _zero(z, _):
            pltpu.async_copy(
                zeros_v, acc.at[pl.ds(tslice + z * ZB, ZB)], sem_z)
            return 0

        def _zero_wait(z, _):
            pltpu.make_async_copy(
                zeros_v, acc.at[pl.ds(tslice + z * ZB, ZB)], sem_z).wait()
            return 0

        lax.fori_loop(0, npieces, _zero, 0)
        lax.fori_loop(0, npieces, _zero_wait, 0)
        plsc.subcore_barrier()

        pltpu.async_copy(
            flat_hbm.at[pl.ds(tile_base, BLK)], idx_buf0, sem_in)
        pltpu.async_copy(
            vals_hbm.at[pl.ds(tile_base, BLK)], val_buf0, sem_in)

        def _scan_block(args, b, parity):
            cnt, drained, inflight = args
            ib = idx_buf0 if parity == 0 else idx_buf1
            vb = val_buf0 if parity == 0 else val_buf1
            nib = idx_buf1 if parity == 0 else idx_buf0
            nvb = val_buf1 if parity == 0 else val_buf0
            off_b = tile_base + b * BLK
            pltpu.make_async_copy(
                flat_hbm.at[pl.ds(off_b, BLK)], ib, sem_in).wait()
            pltpu.make_async_copy(
                vals_hbm.at[pl.ds(off_b, BLK)], vb, sem_in).wait()

            @pl.when(b + 1 < NBLK)
            def _():
                off = tile_base + (b + 1) * BLK
                pltpu.async_copy(flat_hbm.at[pl.ds(off, BLK)], nib, sem_in)
                pltpu.async_copy(vals_hbm.at[pl.ds(off, BLK)], nvb, sem_in)

            base_vec = jnp.zeros((LANES,), jnp.int32) + base
            cbound = plsc.bitcast(
                jnp.zeros((LANES,), jnp.int32) + bound, jnp.uint32)

            def _vec(v, cnt_vec):
                fl = ib[pl.ds(v * LANES, LANES)]
                vv = vb[pl.ds(v * LANES, LANES)]
                t = fl - base_vec
                mask = plsc.bitcast(t, jnp.uint32) < cbound
                cums = plsc.cumsum(mask.astype(jnp.int32))
                pos = cnt_vec + cums
                rows = (pos >> 7) & (NR - 1)
                cols = pos & (ROW - 1)
                plsc.store_scatter(stage_idx, [rows, cols], t, mask=mask)
                plsc.store_scatter(stage_val, [rows, cols], vv, mask=mask)
                pc = plsc.all_reduce_population_count(mask)
                return cnt_vec + pc

            cnt_vec0 = jnp.zeros((LANES,), jnp.int32) + (cnt - 1)
            cnt_vec = plsc.parallel_loop(
                0, BLK // LANES, 1, unroll=4, carry=cnt_vec0)(_vec)
            cnt = cnt_vec[0] + 1

            full = cnt >> 7

            def _drain(j, _):
                row = j & (NR - 1)
                pltpu.async_copy(
                    stage_val.at[row], acc.at[stage_idx.at[row]], sem_d,
                    add=True)
                return 0

            lax.fori_loop(drained, full, _drain, 0)
            inflight = inflight + (full - drained)

            def _wait_one(j, _):
                _dummy_wait()
                return 0

            nwait = jnp.maximum(inflight - INFLIGHT_CAP, 0)
            lax.fori_loop(0, nwait, _wait_one, 0)
            inflight = inflight - nwait
            return cnt, full, inflight

        def _block_pair(i, args):
            args = _scan_block(args, 2 * i, 0)
            args = _scan_block(args, 2 * i + 1, 1)
            return args

        cnt, drained, inflight = lax.fori_loop(
            0, NBLK // 2, _block_pair, (jnp.int32(0), jnp.int32(0),
                                        jnp.int32(0)))

        col = cnt & (ROW - 1)
        row_last = (cnt >> 7) & (NR - 1)

        @pl.when(col != 0)
        def _():
            rvec = jnp.zeros((LANES,), jnp.int32) + row_last
            zidx = jnp.zeros((LANES,), jnp.int32)
            cvec = jnp.zeros((LANES,), jnp.int32) + col
            for k in range(ROW // LANES):
                lane = lax.iota(jnp.int32, LANES) + (k * LANES) + cvec
                m = lane < ROW
                plsc.store_scatter(stage_idx, [rvec, lane], zidx, mask=m)
                plsc.store_scatter(stage_val, [rvec, lane], zvec, mask=m)

        full_end = (cnt + (ROW - 1)) >> 7

        def _drain_tail(j, _):
            row = j & (NR - 1)
            pltpu.async_copy(
                stage_val.at[row], acc.at[stage_idx.at[row]], sem_d, add=True)
            return 0

        lax.fori_loop(drained, full_end, _drain_tail, 0)
        inflight = inflight + (full_end - drained)

        def _wait_all(j, _):
            _dummy_wait()
            return 0

        lax.fori_loop(0, inflight, _wait_all, 0)
        plsc.subcore_barrier()

        def _flush(z, _):
            pltpu.async_copy(
                acc.at[pl.ds(tslice + z * ZB, ZB)],
                out_hbm.at[pl.ds(base + tslice + z * ZB, ZB)], sem_z)
            return 0

        def _flush_wait(z, _):
            pltpu.make_async_copy(
                acc.at[pl.ds(tslice + z * ZB, ZB)],
                out_hbm.at[pl.ds(base + tslice + z * ZB, ZB)], sem_z).wait()
            return 0

        lax.fori_loop(0, npieces, _flush, 0)
        lax.fori_loop(0, npieces, _flush_wait, 0)
        plsc.subcore_barrier()
        return 0

    lax.fori_loop(0, ROUNDS, _round, 0)


_densify_call = pl.kernel(
    _sc_body,
    out_type=jax.ShapeDtypeStruct((NN,), jnp.float32),
    mesh=plsc.VectorSubcoreMesh(
        core_axis_name="c", subcore_axis_name="s", num_cores=NC,
        num_subcores=NS),
    scratch_types=[
        pltpu.VMEM_SHARED((C,), jnp.float32),
        pltpu.VMEM((BLK,), jnp.int32),
        pltpu.VMEM((BLK,), jnp.int32),
        pltpu.VMEM((BLK,), jnp.float32),
        pltpu.VMEM((BLK,), jnp.float32),
        pltpu.VMEM((NR, ROW), jnp.int32),
        pltpu.VMEM((NR, ROW), jnp.float32),
        pltpu.VMEM((ZB,), jnp.float32),
        pltpu.SemaphoreType.DMA,
        pltpu.SemaphoreType.DMA,
        pltpu.SemaphoreType.DMA,
    ],
    compiler_params=pltpu.CompilerParams(needs_layout_passes=False),
    cost_estimate=pl.CostEstimate(
        flops=2 * NNZ_PAD * ROUNDS,
        bytes_accessed=8 * NNZ_PAD * ROUNDS * NC + 4 * NN,
        transcendentals=0),
)


def _build_wt(rows, cols, vals):
    flat = cols * N + rows
    flat = jnp.concatenate(
        [flat, jnp.full((NNZ_PAD - NNZ,), PAD_IDX, jnp.int32)])
    v = jnp.concatenate(
        [vals, jnp.zeros((NNZ_PAD - NNZ,), jnp.float32)])
    return _densify_call(flat, v).reshape(N, N)


BM = 1024
BN = 1024
BK = 2048


def _mm_body(apply_tanh, out_dtype, x_ref, w_ref, b_ref, o_ref, acc_ref):
    k = pl.program_id(1)

    @pl.when(k == 0)
    def _():
        acc_ref[...] = jnp.zeros_like(acc_ref)

    acc_ref[...] += jnp.dot(
        x_ref[...], w_ref[...].astype(jnp.bfloat16),
        preferred_element_type=jnp.float32)

    @pl.when(k == pl.num_programs(1) - 1)
    def _():
        y = acc_ref[...] + b_ref[...].astype(jnp.float32)
        if apply_tanh:
            y = jnp.tanh(y)
        o_ref[...] = y.astype(out_dtype)


def _mm(x_bf16, wt, bias, apply_tanh, out_dtype):
    b2 = bias.reshape(1, N)
    grid = (N // BN, N // BK)
    return pl.pallas_call(
        functools.partial(_mm_body, apply_tanh, out_dtype),
        grid=grid,
        in_specs=[
            pl.BlockSpec((BM, BK), lambda n, k: (0, k)),
            pl.BlockSpec((BK, BN), lambda n, k: (k, n)),
            pl.BlockSpec((1, BN), lambda n, k: (0, n)),
        ],
        out_specs=pl.BlockSpec((BM, BN), lambda n, k: (0, n)),
        out_shape=jax.ShapeDtypeStruct((BM, N), out_dtype),
        scratch_shapes=[pltpu.VMEM((BM, BN), jnp.float32)],
        compiler_params=pltpu.CompilerParams(
            dimension_semantics=("parallel", "arbitrary")),
    )(x_bf16, wt, b2)


def kernel(x, rows0, cols0, vals0, bias0, rows1, cols1, vals1, bias1,
           rows2, cols2, vals2, bias2):
    wt0 = _build_wt(rows0, cols0, vals0)
    wt1 = _build_wt(rows1, cols1, vals1)
    wt2 = _build_wt(rows2, cols2, vals2)
    h = _mm(x.astype(jnp.bfloat16), wt0, bias0, True, jnp.bfloat16)
    h = _mm(h, wt1, bias1, True, jnp.bfloat16)
    return _mm(h, wt2, bias2, False, jnp.float32)
